# Initial kernel scaffold; baseline (speedup 1.0000x reference)
import jax
import jax.numpy as jnp
from jax.experimental import pallas as pl


def kernel(video_emb, audio_emb, y, epoch, view1_mem, view2_mem):
    raise NotImplementedError("write your pallas kernel here")


# speedup vs baseline: 4.5936x; 4.5936x over previous
import functools

import jax
import jax.numpy as jnp
from jax import lax
from jax.experimental import pallas as pl
from jax.experimental.pallas import tpu as pltpu
from jax.experimental.pallas import tpu_sc as plsc

MEM = 100000
D = 64
K = 1024
B = 1024
TEMP = 0.07

CM = 2048
NCHUNK = (MEM + CM - 1) // CM

PADW = 1152
NIDX = PADW // 128

NC = 2
NS = 16
NW = NC * NS
ROWS_PER_W = B // NW


def _prep_body(v_ref, a_ref, y_ref, base_ref, qv_ref, qa_ref, idx_ref):
    inv_t = 1.0 / TEMP
    v = v_ref[...]
    a = a_ref[...]
    vn = jnp.maximum(jnp.sum(v * v, axis=1, keepdims=True), 1e-24)
    an = jnp.maximum(jnp.sum(a * a, axis=1, keepdims=True), 1e-24)
    qv_ref[...] = v * lax.rsqrt(vn) * inv_t
    qa_ref[...] = a * lax.rsqrt(an) * inv_t

    y = y_ref[...]
    base = base_ref[...]
    row = lax.broadcasted_iota(jnp.int32, (B, K), 0) * MEM
    neg = base + jnp.where(base >= y, 1, 0).astype(jnp.int32) + row
    rowp = lax.broadcasted_iota(jnp.int32, (B, PADW - K), 0) * MEM
    pos = y + rowp
    idx_ref[...] = jnp.concatenate([neg, pos], axis=1)


def _mm_body(qv_ref, qa_ref, m2_ref, m1_ref, s2_ref, s1_ref):
    dn = (((1,), (1,)), ((), ()))
    s2_ref[...] = lax.dot_general(qv_ref[...], m2_ref[...], dn,
                                  preferred_element_type=jnp.float32)
    s1_ref[...] = lax.dot_general(qa_ref[...], m1_ref[...], dn,
                                  preferred_element_type=jnp.float32)


_sc_mesh = plsc.VectorSubcoreMesh(core_axis_name="c", subcore_axis_name="s")


@functools.partial(
    pl.kernel,
    mesh=_sc_mesh,
    out_type=[jax.ShapeDtypeStruct((B * PADW,), jnp.float32),
              jax.ShapeDtypeStruct((B * PADW,), jnp.float32)],
    scratch_types=[
        pltpu.VMEM((ROWS_PER_W * PADW,), jnp.int32),
        pltpu.VMEM((ROWS_PER_W * PADW,), jnp.float32),
        pltpu.VMEM((ROWS_PER_W * PADW,), jnp.float32),
        pltpu.SemaphoreType.DMA,
    ],
)
def _sc_gather(s2_hbm, s1_hbm, idx_hbm, ov_hbm, oa_hbm, idx_v, gv_v, ga_v, sem):
    wid = lax.axis_index("s") * NC + lax.axis_index("c")
    e0 = wid * ROWS_PER_W * PADW
    pltpu.sync_copy(idx_hbm.at[pl.ds(e0, ROWS_PER_W * PADW)], idx_v)
    copies = []
    for i in range(ROWS_PER_W):
        sl = pl.ds(i * PADW, PADW)
        copies.append(pltpu.async_copy(s2_hbm.at[idx_v.at[sl]], gv_v.at[sl], sem))
        copies.append(pltpu.async_copy(s1_hbm.at[idx_v.at[sl]], ga_v.at[sl], sem))
    for cp in copies:
        cp.wait()
    pltpu.sync_copy(gv_v, ov_hbm.at[pl.ds(e0, ROWS_PER_W * PADW)])
    pltpu.sync_copy(ga_v, oa_hbm.at[pl.ds(e0, ROWS_PER_W * PADW)])


def kernel(video_emb, audio_emb, y, epoch, view1_mem, view2_mem):
    y2d = y.astype(jnp.int32).reshape(B, 1)
    base = jax.random.randint(jax.random.key(42), (B, K), 0, MEM - 1,
                              dtype=jnp.int32)

    qv, qa, idx = pl.pallas_call(
        _prep_body,
        out_shape=[jax.ShapeDtypeStruct((B, D), jnp.float32),
                   jax.ShapeDtypeStruct((B, D), jnp.float32),
                   jax.ShapeDtypeStruct((B, PADW), jnp.int32)],
    )(video_emb, audio_emb, y2d, base)

    S2, S1 = pl.pallas_call(
        _mm_body,
        grid=(NCHUNK,),
        in_specs=[
            pl.BlockSpec((B, D), lambda i: (0, 0)),
            pl.BlockSpec((B, D), lambda i: (0, 0)),
            pl.BlockSpec((CM, D), lambda i: (i, 0)),
            pl.BlockSpec((CM, D), lambda i: (i, 0)),
        ],
        out_specs=[
            pl.BlockSpec((B, CM), lambda i: (0, i)),
            pl.BlockSpec((B, CM), lambda i: (0, i)),
        ],
        out_shape=[jax.ShapeDtypeStruct((B, MEM), jnp.float32)] * 2,
    )(qv, qa, view2_mem, view1_mem)

    ov, oa = _sc_gather(S2.reshape(-1), S1.reshape(-1),
                        idx.reshape(-1))
    ov = ov.reshape(B, PADW)
    oa = oa.reshape(B, PADW)
    return jnp.concatenate(
        [ov[:, K:K + 1], ov[:, :K], oa[:, K:K + 1], oa[:, :K]], axis=1)
